# baseline (speedup 1.0000x reference)
import functools

import jax
import jax.numpy as jnp
from jax import lax
from jax.experimental import pallas as pl
from jax.experimental.pallas import tpu as pltpu
from jax.experimental.pallas import tpu_sc as plsc

B = 16384
D = 64
HIDDEN = 100
NC = 2
NS = 16
NW = NC * NS
BPW = B // NW
PPW = BPW // 2
L = 16
TR = 8
CHB = 16
NCHB = BPW // CHB
NSLOT = 4


def _sc_gather(xs, tables):
    n = len(xs)
    mesh = plsc.VectorSubcoreMesh(core_axis_name="c", subcore_axis_name="s")

    @functools.partial(
        pl.kernel,
        mesh=mesh,
        out_type=[jax.ShapeDtypeStruct((B // 2, 2 * D), jnp.float32)] * n,
        scratch_types=[
            pltpu.VMEM((BPW,), jnp.int32),
            pltpu.VMEM((NSLOT, CHB, TR, D), jnp.float32),
            pltpu.VMEM((PPW, 2 * D), jnp.float32),
            pltpu.SemaphoreType.DMA,
            pltpu.SemaphoreType.DMA,
        ],
    )
    def gather_kernel(*refs):
        xhs = refs[:n]
        tbls = refs[n:2 * n]
        ohs = refs[2 * n:3 * n]
        iv, gbuf, stag, gsem, wsem = refs[3 * n:]
        wid = lax.axis_index("s") * NC + lax.axis_index("c")
        base = wid * BPW
        obase = wid * PPW

        for xh, tbl, oh in zip(xhs, tbls, ohs):
            pltpu.sync_copy(xh.at[pl.ds(base, BPW)], iv)

            def fire(t, slot):
                def grp(gi, _):
                    vec = iv[pl.ds(t * CHB + gi * L, L)]
                    for lane in range(L):
                        g = lax.shift_right_logical(vec[lane], 3)
                        pltpu.async_copy(
                            tbl.at[pl.ds(g, 1)],
                            gbuf.at[slot].at[pl.ds(gi * L + lane, 1)], gsem)
                    return _

                lax.fori_loop(0, CHB // L, grp, 0)

            def drain(slot):
                pltpu.make_async_copy(
                    tbl.at[pl.ds(0, CHB)], gbuf.at[slot], gsem).wait()

            def select(t, slot):
                gb = gbuf.at[slot]

                def sgrp(gi, _):
                    vec = iv[pl.ds(t * CHB + gi * L, L)]
                    sub = vec & (TR - 1)
                    for li in range(L // 2):
                        row = gi * L + 2 * li
                        r0 = sub[2 * li]
                        r1 = sub[2 * li + 1]
                        p = t * (CHB // 2) + gi * (L // 2) + li
                        for q in range(D // L):
                            sl = pl.ds(q * L, L)
                            stag[p, sl] = gb[row, r0, sl]
                            stag[p, pl.ds(D + q * L, L)] = gb[row + 1, r1, sl]
                    return _

                lax.fori_loop(0, CHB // L, sgrp, 0)

            for pre in range(NSLOT - 1):
                fire(pre, pre)

            def step(t, _):
                @pl.when(t + NSLOT - 1 < NCHB)
                def _prefetch():
                    fire(t + NSLOT - 1, lax.rem(t + NSLOT - 1, NSLOT))

                slot = lax.rem(t, NSLOT)
                drain(slot)
                select(t, slot)
                rows = CHB // 2
                pltpu.async_copy(
                    stag.at[pl.ds(t * rows, rows)],
                    oh.at[pl.ds(obase + t * rows, rows)], wsem)
                return _

            lax.fori_loop(0, NCHB, step, 0)
            pltpu.make_async_copy(
                stag, oh.at[pl.ds(obase, PPW)], wsem).wait()

    return gather_kernel(*xs, *[t.reshape(-1, TR, D) for t in tables])


RB2 = 1024


def _mlp_part_kernel(e2r, e3r, w1r, b1r, outr):
    def head(lo):
        sl = slice(lo, lo + D)
        h = jnp.dot(e2r[:, sl], w1r[D:2 * D, :],
                    preferred_element_type=jnp.float32)
        h = h + jnp.dot(e3r[:, sl], w1r[2 * D:3 * D, :],
                        preferred_element_type=jnp.float32)
        return h + b1r[...]

    outr[...] = jnp.concatenate([head(0), head(D)], axis=1)


def _mlp_part(e2, e3, W1, b1):
    return pl.pallas_call(
        _mlp_part_kernel,
        grid=(B // 2 // RB2,),
        in_specs=[
            pl.BlockSpec((RB2, 2 * D), lambda i: (i, 0)),
            pl.BlockSpec((RB2, 2 * D), lambda i: (i, 0)),
            pl.BlockSpec((3 * D, HIDDEN), lambda i: (0, 0)),
            pl.BlockSpec((1, HIDDEN), lambda i: (0, 0)),
        ],
        out_specs=pl.BlockSpec((RB2, 2 * HIDDEN), lambda i: (i, 0)),
        out_shape=jax.ShapeDtypeStruct((B // 2, 2 * HIDDEN), jnp.float32),
    )(e2, e3, W1, b1)


def _mlp_final_kernel(e1r, hpr, w1r, w2r, b2r, outr):
    def head(lo, hs):
        h = hpr[:, hs] + jnp.dot(e1r[:, lo:lo + D], w1r[0:D, :],
                                 preferred_element_type=jnp.float32)
        h = jnp.maximum(h, 0.0)
        o = jnp.dot(h, w2r[...], preferred_element_type=jnp.float32) + b2r[...]
        return 1.0 / (1.0 + jnp.exp(-o))

    outr[...] = jnp.concatenate(
        [head(0, slice(0, HIDDEN)), head(D, slice(HIDDEN, 2 * HIDDEN))],
        axis=1)


def _mlp_final(e1, hp, W1, W2, b2):
    return pl.pallas_call(
        _mlp_final_kernel,
        grid=(B // 2 // RB2,),
        in_specs=[
            pl.BlockSpec((RB2, 2 * D), lambda i: (i, 0)),
            pl.BlockSpec((RB2, 2 * HIDDEN), lambda i: (i, 0)),
            pl.BlockSpec((3 * D, HIDDEN), lambda i: (0, 0)),
            pl.BlockSpec((HIDDEN, 1), lambda i: (0, 0)),
            pl.BlockSpec((1, 1), lambda i: (0, 0)),
        ],
        out_specs=pl.BlockSpec((RB2, 2), lambda i: (i, 0)),
        out_shape=jax.ShapeDtypeStruct((B // 2, 2), jnp.float32),
    )(e1, hp, W1, W2, b2)


def kernel(x1, x2, x3, user_embed, movie_embed, category_embed, W1, b1, W2, b2):
    x1 = x1.astype(jnp.int32)
    x2 = x2.astype(jnp.int32)
    x3 = x3.astype(jnp.int32)
    e2, e3 = _sc_gather([x2, x3], [movie_embed, category_embed])
    hp = _mlp_part(e2, e3, W1, b1.reshape(1, HIDDEN))
    (e1,) = _sc_gather([x1], [user_embed])
    out = _mlp_final(e1, hp, W1, W2, b2.reshape(1, 1))
    return out.reshape(B, 1)


# speedup vs baseline: 1.0171x; 1.0171x over previous
import functools

import jax
import jax.numpy as jnp
from jax import lax
from jax.experimental import pallas as pl
from jax.experimental.pallas import tpu as pltpu
from jax.experimental.pallas import tpu_sc as plsc

B = 16384
D = 64
HIDDEN = 100
NC = 2
NS = 16
NW = NC * NS
BPW = B // NW
PPW = BPW // 2
L = 16
TR = 8
CHB = 16
NCHB = BPW // CHB
NSLOT = 4


def _sc_gather(x1, x2, x3, user_t, movie_t, category_t):
    mesh = plsc.VectorSubcoreMesh(core_axis_name="c", subcore_axis_name="s")

    @functools.partial(
        pl.kernel,
        mesh=mesh,
        out_type=[jax.ShapeDtypeStruct((B // 2, 2 * D), jnp.float32)] * 3,
        scratch_types=[
            pltpu.VMEM((BPW,), jnp.int32),
            pltpu.VMEM((NSLOT, CHB, TR, D), jnp.float32),
            pltpu.VMEM((PPW, 2 * D), jnp.float32),
            pltpu.SemaphoreType.DMA,
            pltpu.SemaphoreType.DMA,
        ],
    )
    def gather_kernel(x1h, x2h, x3h, uh, mh, ch, o1h, o2h, o3h,
                      iv, gbuf, stag, gsem, wsem):
        wid = lax.axis_index("s") * NC + lax.axis_index("c")
        base = wid * BPW
        obase = wid * PPW

        for xh, tbl, oh in ((x1h, uh, o1h), (x2h, mh, o2h), (x3h, ch, o3h)):
            pltpu.sync_copy(xh.at[pl.ds(base, BPW)], iv)

            def fire(t, slot):
                def grp(gi, _):
                    vec = iv[pl.ds(t * CHB + gi * L, L)]
                    for lane in range(L):
                        g = lax.shift_right_logical(vec[lane], 3)
                        pltpu.async_copy(
                            tbl.at[pl.ds(g, 1)],
                            gbuf.at[slot].at[pl.ds(gi * L + lane, 1)], gsem)
                    return _

                lax.fori_loop(0, CHB // L, grp, 0)

            def drain(slot):
                pltpu.make_async_copy(
                    tbl.at[pl.ds(0, CHB)], gbuf.at[slot], gsem).wait()

            def select(t, slot):
                gb = gbuf.at[slot]

                def sgrp(gi, _):
                    vec = iv[pl.ds(t * CHB + gi * L, L)]
                    sub = vec & (TR - 1)
                    for li in range(L // 2):
                        row = gi * L + 2 * li
                        r0 = sub[2 * li]
                        r1 = sub[2 * li + 1]
                        p = t * (CHB // 2) + gi * (L // 2) + li
                        for q in range(D // L):
                            sl = pl.ds(q * L, L)
                            stag[p, sl] = gb[row, r0, sl]
                            stag[p, pl.ds(D + q * L, L)] = gb[row + 1, r1, sl]
                    return _

                lax.fori_loop(0, CHB // L, sgrp, 0)

            for pre in range(NSLOT - 1):
                fire(pre, pre)

            def step(t, _):
                @pl.when(t + NSLOT - 1 < NCHB)
                def _prefetch():
                    fire(t + NSLOT - 1, lax.rem(t + NSLOT - 1, NSLOT))

                slot = lax.rem(t, NSLOT)
                drain(slot)
                select(t, slot)
                rows = CHB // 2
                pltpu.async_copy(
                    stag.at[pl.ds(t * rows, rows)],
                    oh.at[pl.ds(obase + t * rows, rows)], wsem)
                return _

            lax.fori_loop(0, NCHB, step, 0)
            pltpu.make_async_copy(
                stag, oh.at[pl.ds(obase, PPW)], wsem).wait()

    return gather_kernel(
        x1, x2, x3,
        user_t.reshape(-1, TR, D),
        movie_t.reshape(-1, TR, D),
        category_t.reshape(-1, TR, D),
    )


RB2 = 1024


def _mlp_kernel(e1r, e2r, e3r, w1r, b1r, w2r, b2r, outr):
    def head(lo):
        sl = slice(lo, lo + D)
        h = jnp.dot(e1r[:, sl], w1r[0:D, :], preferred_element_type=jnp.float32)
        h = h + jnp.dot(e2r[:, sl], w1r[D:2 * D, :],
                        preferred_element_type=jnp.float32)
        h = h + jnp.dot(e3r[:, sl], w1r[2 * D:3 * D, :],
                        preferred_element_type=jnp.float32)
        h = jnp.maximum(h + b1r[...], 0.0)
        o = jnp.dot(h, w2r[...], preferred_element_type=jnp.float32) + b2r[...]
        return 1.0 / (1.0 + jnp.exp(-o))

    outr[...] = jnp.concatenate([head(0), head(D)], axis=1)


def _mlp(e1, e2, e3, W1, b1, W2, b2):
    grid = (B // 2 // RB2,)
    return pl.pallas_call(
        _mlp_kernel,
        grid=grid,
        in_specs=[
            pl.BlockSpec((RB2, 2 * D), lambda i: (i, 0)),
            pl.BlockSpec((RB2, 2 * D), lambda i: (i, 0)),
            pl.BlockSpec((RB2, 2 * D), lambda i: (i, 0)),
            pl.BlockSpec((3 * D, HIDDEN), lambda i: (0, 0)),
            pl.BlockSpec((1, HIDDEN), lambda i: (0, 0)),
            pl.BlockSpec((HIDDEN, 1), lambda i: (0, 0)),
            pl.BlockSpec((1, 1), lambda i: (0, 0)),
        ],
        out_specs=pl.BlockSpec((RB2, 2), lambda i: (i, 0)),
        out_shape=jax.ShapeDtypeStruct((B // 2, 2), jnp.float32),
    )(e1, e2, e3, W1, b1, W2, b2)


def kernel(x1, x2, x3, user_embed, movie_embed, category_embed, W1, b1, W2, b2):
    x1 = x1.astype(jnp.int32)
    x2 = x2.astype(jnp.int32)
    x3 = x3.astype(jnp.int32)
    e1, e2, e3 = _sc_gather(
        x1, x2, x3, user_embed, movie_embed, category_embed)
    out = _mlp(e1, e2, e3, W1, b1.reshape(1, HIDDEN), W2, b2.reshape(1, 1))
    return out.reshape(B, 1)
